# baseline (speedup 1.0000x reference)
import jax
import jax.numpy as jnp
from jax import lax
from jax.experimental import pallas as pl
from jax.experimental.pallas import tpu as pltpu
from jax.experimental.pallas import tpu_sc as plsc

SIZE = 8388608
DIM = 64
N_CHUNKS = 8
BATCH = 16384
MERSENNE = (1 << 31) - 1
OUT_MASK = (SIZE // 2) - 1

L = 16
RW = 64
RWB = 6
VROWS = SIZE // RW
RPS = DIM // RW + 1

NC = 2
NS = 16
NW = NC * NS
B_W = BATCH // NW
G = 32
NG = B_W // G
SLICES_G = G * N_CHUNKS
ROWS_G = (RPS + 1) * SLICES_G
WOFF = RPS * SLICES_G
N_SLICES = B_W * N_CHUNKS
SCALE = float((N_CHUNKS * DIM) ** 0.5 / N_CHUNKS)


def _iota():
    return lax.broadcasted_iota(jnp.int32, (L,), 0)


def _hash_mersenne(x, a, b):
    m = jnp.uint32(MERSENNE)
    x0 = x & jnp.uint32(0xFFFF)
    x1 = x >> jnp.uint32(16)
    a0 = a & jnp.uint32(0xFFFF)
    a1 = a >> jnp.uint32(16)
    low = x0 * a0
    mid = x1 * a0 + x0 * a1
    hi = x1 * a1
    m0 = mid & jnp.uint32(0x7FFF)
    m1 = mid >> jnp.uint32(15)
    l0 = low & m
    l1 = low >> jnp.uint32(31)
    t1 = (m0 << jnp.uint32(16)) + l0
    t1 = (t1 & m) + (t1 >> jnp.uint32(31))
    t2 = t1 + (hi << jnp.uint32(1)) + m1 + l1
    t2 = (t2 & m) + (t2 >> jnp.uint32(31))
    s = t2 + b
    s = (s & m) + (s >> jnp.uint32(31))
    s = jnp.where(s >= m, s - m, s)
    return s & jnp.uint32(OUT_MASK)


def _body(x_hbm, ca0_hbm, cb0_hbm, ca1_hbm, cb1_hbm, table_hbm,
          out_hbm, x_v, ca0_v, cb0_v, ca1_v, cb1_v, rowidx_v, meta_v,
          rows_a, rows_b, out_a, out_b, sem_a, sem_b, sem_out):
    wid = lax.axis_index("s") * NC + lax.axis_index("c")
    base_b = wid * B_W

    pltpu.sync_copy(x_hbm.at[pl.ds(base_b, B_W)], x_v)
    pltpu.sync_copy(ca0_hbm, ca0_v)
    pltpu.sync_copy(cb0_hbm, cb0_v)
    pltpu.sync_copy(ca1_hbm, ca1_v)
    pltpu.sync_copy(cb1_hbm, cb1_v)

    ca0 = plsc.bitcast(ca0_v[...], jnp.uint32)
    cb0 = plsc.bitcast(cb0_v[...], jnp.uint32)
    ca1 = plsc.bitcast(ca1_v[...], jnp.uint32)
    cb1 = plsc.bitcast(cb1_v[...], jnp.uint32)
    iota = _iota()

    def hash_step(p, carry):
        xi = plsc.load_gather(x_v, [2 * p + (iota >> 3)])
        xu = plsc.bitcast(xi, jnp.uint32)
        h0 = plsc.bitcast(_hash_mersenne(xu, ca0, cb0), jnp.int32)
        h1 = plsc.bitcast(_hash_mersenne(xu, ca1, cb1), jnp.int32)
        q = h0 >> RWB
        s = 16 * p + iota
        sl = s & (SLICES_G - 1)
        gbase = (s >> 8) * ROWS_G
        for i in range(RPS):
            plsc.store_scatter(rowidx_v, [gbase + RPS * sl + i], q + i)
        plsc.store_scatter(rowidx_v, [gbase + WOFF + sl], h1 >> RWB)
        meta_v[pl.ds(16 * p, L)] = (
            ((h1 & (RW - 1)) << 16) | (RW * RPS * sl + (h0 & (RW - 1))))
        return carry

    def hash_group(g):
        lax.fori_loop(jnp.int32(g * (SLICES_G // L)),
                      jnp.int32((g + 1) * (SLICES_G // L)), hash_step, 0)

    def start(g, rows_v, sem):
        handles = []
        for k in range(ROWS_G // 128):
            handles.append(pltpu.async_copy(
                table_hbm.at[rowidx_v.at[pl.ds(g * ROWS_G + k * 128, 128)]],
                rows_v.at[pl.ds(k * 128, 128)], sem))
        return handles

    bufs = [(rows_a, sem_a), (rows_b, sem_b)]
    out_bufs = [out_a, out_b]
    hash_group(0)
    pend = {0: start(0, *bufs[0])}
    out_pend = [None, None]

    for g in range(NG):
        b = g & 1
        if g + 1 < NG:
            hash_group(g + 1)
            pend[g + 1] = start(g + 1, *bufs[(g + 1) & 1])
        for h in pend.pop(g):
            h.wait()
        rows_v, _ = bufs[b]
        out_g = out_bufs[b]
        if out_pend[b] is not None:
            out_pend[b].wait()
            out_pend[b] = None

        def item_step(i, carry2, rows_v=rows_v, out_g=out_g, g=g):
            accs = [jnp.zeros((L,), jnp.float32) for _ in range(DIM // L)]
            for c in range(N_CHUNKS):
                sl = i * N_CHUNKS + c
                meta = plsc.load_gather(
                    meta_v, [jnp.full((L,), g * SLICES_G, jnp.int32) + sl])
                wc = meta >> 16
                base = (meta & 0xFFFF) + iota
                w = plsc.load_gather(
                    rows_v, [jnp.full((L,), WOFF + sl, jnp.int32), wc])
                for v in range(DIM // L):
                    flat = base + v * L
                    vec = plsc.load_gather(rows_v, [flat >> RWB, flat & (RW - 1)])
                    accs[v] = accs[v] + vec * w
            for v in range(DIM // L):
                out_g[i, pl.ds(v * L, L)] = accs[v] * SCALE
            return carry2

        lax.fori_loop(jnp.int32(0), jnp.int32(G), item_step, 0)
        out_pend[b] = pltpu.async_copy(
            out_g, out_hbm.at[pl.ds(base_b + g * G, G)], sem_out)

    for op in out_pend:
        if op is not None:
            op.wait()


@jax.jit
def _robe(x32, table16, ca0, cb0, ca1, cb1):
    mesh = plsc.VectorSubcoreMesh(core_axis_name="c", subcore_axis_name="s")
    f = pl.kernel(
        _body,
        mesh=mesh,
        out_type=jax.ShapeDtypeStruct((BATCH, DIM), jnp.float32),
        compiler_params=pltpu.CompilerParams(
            needs_layout_passes=False, use_tc_tiling_on_sc=False),
        scratch_types=[
            pltpu.VMEM((B_W,), jnp.int32),
            pltpu.VMEM((L,), jnp.int32),
            pltpu.VMEM((L,), jnp.int32),
            pltpu.VMEM((L,), jnp.int32),
            pltpu.VMEM((L,), jnp.int32),
            pltpu.VMEM((NG * ROWS_G,), jnp.int32),
            pltpu.VMEM((N_SLICES,), jnp.int32),
            pltpu.VMEM((ROWS_G, RW), jnp.float32),
            pltpu.VMEM((ROWS_G, RW), jnp.float32),
            pltpu.VMEM((G, DIM), jnp.float32),
            pltpu.VMEM((G, DIM), jnp.float32),
            pltpu.SemaphoreType.DMA,
            pltpu.SemaphoreType.DMA,
            pltpu.SemaphoreType.DMA,
        ],
    )
    return f(x32, ca0, cb0, ca1, cb1, table16)


def kernel(x, table, coeffs0, coeffs1):
    x32 = x.astype(jnp.int32)
    table16 = table.reshape(VROWS, RW)
    ca0 = jnp.tile(coeffs0[:, 0].astype(jnp.int32), 2)
    cb0 = jnp.tile(coeffs0[:, 1].astype(jnp.int32), 2)
    ca1 = jnp.tile(coeffs1[:, 0].astype(jnp.int32), 2)
    cb1 = jnp.tile(coeffs1[:, 1].astype(jnp.int32), 2)
    return _robe(x32, table16, ca0, cb0, ca1, cb1)


# speedup vs baseline: 1.0566x; 1.0566x over previous
import jax
import jax.numpy as jnp
from jax import lax
from jax.experimental import pallas as pl
from jax.experimental.pallas import tpu as pltpu
from jax.experimental.pallas import tpu_sc as plsc

SIZE = 8388608
DIM = 64
N_CHUNKS = 8
BATCH = 16384
MERSENNE = (1 << 31) - 1
OUT_MASK = (SIZE // 2) - 1

L = 16
RW = 16
RWB = 4
VROWS = SIZE // RW
RPS = DIM // RW + 1

NC = 2
NS = 16
NW = NC * NS
B_W = BATCH // NW
G = 32
NG = B_W // G
SLICES_G = G * N_CHUNKS
ROWS_G = (RPS + 1) * SLICES_G
WOFF = RPS * SLICES_G
N_SLICES = B_W * N_CHUNKS
SCALE = float((N_CHUNKS * DIM) ** 0.5 / N_CHUNKS)


def _iota():
    return lax.broadcasted_iota(jnp.int32, (L,), 0)


def _hash_mersenne(x, a, b):
    m = jnp.uint32(MERSENNE)
    x0 = x & jnp.uint32(0xFFFF)
    x1 = x >> jnp.uint32(16)
    a0 = a & jnp.uint32(0xFFFF)
    a1 = a >> jnp.uint32(16)
    low = x0 * a0
    mid = x1 * a0 + x0 * a1
    hi = x1 * a1
    m0 = mid & jnp.uint32(0x7FFF)
    m1 = mid >> jnp.uint32(15)
    l0 = low & m
    l1 = low >> jnp.uint32(31)
    t1 = (m0 << jnp.uint32(16)) + l0
    t1 = (t1 & m) + (t1 >> jnp.uint32(31))
    t2 = t1 + (hi << jnp.uint32(1)) + m1 + l1
    t2 = (t2 & m) + (t2 >> jnp.uint32(31))
    s = t2 + b
    s = (s & m) + (s >> jnp.uint32(31))
    s = jnp.where(s >= m, s - m, s)
    return s & jnp.uint32(OUT_MASK)


def _body(x_hbm, ca0_hbm, cb0_hbm, ca1_hbm, cb1_hbm, table_hbm,
          out_hbm, x_v, ca0_v, cb0_v, ca1_v, cb1_v, rowidx_v, meta_v,
          rows_a, rows_b, out_a, out_b, sem_a, sem_b, sem_out):
    wid = lax.axis_index("s") * NC + lax.axis_index("c")
    base_b = wid * B_W

    pltpu.sync_copy(x_hbm.at[pl.ds(base_b, B_W)], x_v)
    pltpu.sync_copy(ca0_hbm, ca0_v)
    pltpu.sync_copy(cb0_hbm, cb0_v)
    pltpu.sync_copy(ca1_hbm, ca1_v)
    pltpu.sync_copy(cb1_hbm, cb1_v)

    ca0 = plsc.bitcast(ca0_v[...], jnp.uint32)
    cb0 = plsc.bitcast(cb0_v[...], jnp.uint32)
    ca1 = plsc.bitcast(ca1_v[...], jnp.uint32)
    cb1 = plsc.bitcast(cb1_v[...], jnp.uint32)
    iota = _iota()

    def hash_step(p, carry):
        xi = plsc.load_gather(x_v, [2 * p + (iota >> 3)])
        xu = plsc.bitcast(xi, jnp.uint32)
        h0 = plsc.bitcast(_hash_mersenne(xu, ca0, cb0), jnp.int32)
        h1 = plsc.bitcast(_hash_mersenne(xu, ca1, cb1), jnp.int32)
        q = h0 >> RWB
        s = 16 * p + iota
        sl = s & (SLICES_G - 1)
        gbase = (s >> 8) * ROWS_G
        for i in range(RPS):
            plsc.store_scatter(rowidx_v, [gbase + RPS * sl + i], q + i)
        plsc.store_scatter(rowidx_v, [gbase + WOFF + sl], h1 >> RWB)
        meta_v[pl.ds(16 * p, L)] = (
            ((h1 & (RW - 1)) << 16) | (RW * RPS * sl + (h0 & (RW - 1))))
        return carry

    def hash_group(g):
        lax.fori_loop(jnp.int32(g * (SLICES_G // L)),
                      jnp.int32((g + 1) * (SLICES_G // L)), hash_step, 0)

    def start(g, rows_v, sem):
        handles = []
        for k in range(ROWS_G // 128):
            handles.append(pltpu.async_copy(
                table_hbm.at[rowidx_v.at[pl.ds(g * ROWS_G + k * 128, 128)]],
                rows_v.at[pl.ds(k * 128, 128)], sem))
        return handles

    bufs = [(rows_a, sem_a), (rows_b, sem_b)]
    out_bufs = [out_a, out_b]
    hash_group(0)
    pend = {0: start(0, *bufs[0])}
    out_pend = [None, None]

    for g in range(NG):
        b = g & 1
        if g + 1 < NG:
            hash_group(g + 1)
            pend[g + 1] = start(g + 1, *bufs[(g + 1) & 1])
        for h in pend.pop(g):
            h.wait()
        rows_v, _ = bufs[b]
        out_g = out_bufs[b]
        if out_pend[b] is not None:
            out_pend[b].wait()
            out_pend[b] = None

        def item_step(i, carry2, rows_v=rows_v, out_g=out_g, g=g):
            accs = [jnp.zeros((L,), jnp.float32) for _ in range(DIM // L)]
            for c in range(N_CHUNKS):
                sl = i * N_CHUNKS + c
                meta = plsc.load_gather(
                    meta_v, [jnp.full((L,), g * SLICES_G, jnp.int32) + sl])
                wc = meta >> 16
                base = (meta & 0xFFFF) + iota
                w = plsc.load_gather(
                    rows_v, [jnp.full((L,), WOFF + sl, jnp.int32), wc])
                for v in range(DIM // L):
                    flat = base + v * L
                    vec = plsc.load_gather(rows_v, [flat >> RWB, flat & (RW - 1)])
                    accs[v] = accs[v] + vec * w
            for v in range(DIM // L):
                out_g[i, pl.ds(v * L, L)] = accs[v] * SCALE
            return carry2

        lax.fori_loop(jnp.int32(0), jnp.int32(G), item_step, 0)
        out_pend[b] = pltpu.async_copy(
            out_g, out_hbm.at[pl.ds(base_b + g * G, G)], sem_out)

    for op in out_pend:
        if op is not None:
            op.wait()


@jax.jit
def _robe(x32, table16, ca0, cb0, ca1, cb1):
    mesh = plsc.VectorSubcoreMesh(core_axis_name="c", subcore_axis_name="s")
    f = pl.kernel(
        _body,
        mesh=mesh,
        out_type=jax.ShapeDtypeStruct((BATCH, DIM), jnp.float32),
        compiler_params=pltpu.CompilerParams(
            needs_layout_passes=False, use_tc_tiling_on_sc=False),
        scratch_types=[
            pltpu.VMEM((B_W,), jnp.int32),
            pltpu.VMEM((L,), jnp.int32),
            pltpu.VMEM((L,), jnp.int32),
            pltpu.VMEM((L,), jnp.int32),
            pltpu.VMEM((L,), jnp.int32),
            pltpu.VMEM((NG * ROWS_G,), jnp.int32),
            pltpu.VMEM((N_SLICES,), jnp.int32),
            pltpu.VMEM((ROWS_G, RW), jnp.float32),
            pltpu.VMEM((ROWS_G, RW), jnp.float32),
            pltpu.VMEM((G, DIM), jnp.float32),
            pltpu.VMEM((G, DIM), jnp.float32),
            pltpu.SemaphoreType.DMA,
            pltpu.SemaphoreType.DMA,
            pltpu.SemaphoreType.DMA,
        ],
    )
    return f(x32, ca0, cb0, ca1, cb1, table16)


def kernel(x, table, coeffs0, coeffs1):
    x32 = x.astype(jnp.int32)
    table16 = table.reshape(VROWS, RW)
    ca0 = jnp.tile(coeffs0[:, 0].astype(jnp.int32), 2)
    cb0 = jnp.tile(coeffs0[:, 1].astype(jnp.int32), 2)
    ca1 = jnp.tile(coeffs1[:, 0].astype(jnp.int32), 2)
    cb1 = jnp.tile(coeffs1[:, 1].astype(jnp.int32), 2)
    return _robe(x32, table16, ca0, cb0, ca1, cb1)


# speedup vs baseline: 1.1464x; 1.0850x over previous
import jax
import jax.numpy as jnp
from jax import lax
from jax.experimental import pallas as pl
from jax.experimental.pallas import tpu as pltpu
from jax.experimental.pallas import tpu_sc as plsc

SIZE = 8388608
DIM = 64
N_CHUNKS = 8
BATCH = 16384
MERSENNE = (1 << 31) - 1
OUT_MASK = (SIZE // 2) - 1

L = 16
RW = 32
RWB = 5
VROWS = SIZE // RW
RPS = DIM // RW + 1

NC = 2
NS = 16
NW = NC * NS
B_W = BATCH // NW
G = 32
NG = B_W // G
SLICES_G = G * N_CHUNKS
ROWS_G = (RPS + 1) * SLICES_G
WOFF = RPS * SLICES_G
N_SLICES = B_W * N_CHUNKS
SCALE = float((N_CHUNKS * DIM) ** 0.5 / N_CHUNKS)


def _iota():
    return lax.broadcasted_iota(jnp.int32, (L,), 0)


def _hash_mersenne(x, a, b):
    m = jnp.uint32(MERSENNE)
    x0 = x & jnp.uint32(0xFFFF)
    x1 = x >> jnp.uint32(16)
    a0 = a & jnp.uint32(0xFFFF)
    a1 = a >> jnp.uint32(16)
    low = x0 * a0
    mid = x1 * a0 + x0 * a1
    hi = x1 * a1
    m0 = mid & jnp.uint32(0x7FFF)
    m1 = mid >> jnp.uint32(15)
    l0 = low & m
    l1 = low >> jnp.uint32(31)
    t1 = (m0 << jnp.uint32(16)) + l0
    t1 = (t1 & m) + (t1 >> jnp.uint32(31))
    t2 = t1 + (hi << jnp.uint32(1)) + m1 + l1
    t2 = (t2 & m) + (t2 >> jnp.uint32(31))
    s = t2 + b
    s = (s & m) + (s >> jnp.uint32(31))
    s = jnp.where(s >= m, s - m, s)
    return s & jnp.uint32(OUT_MASK)


def _body(x_hbm, ca0_hbm, cb0_hbm, ca1_hbm, cb1_hbm, table_hbm,
          out_hbm, x_v, ca0_v, cb0_v, ca1_v, cb1_v, rowidx_v, meta_v,
          rows_a, rows_b, rows_c, out_a, out_b, sem_a, sem_b, sem_c,
          sem_out):
    wid = lax.axis_index("s") * NC + lax.axis_index("c")
    base_b = wid * B_W

    pltpu.sync_copy(x_hbm.at[pl.ds(base_b, B_W)], x_v)
    pltpu.sync_copy(ca0_hbm, ca0_v)
    pltpu.sync_copy(cb0_hbm, cb0_v)
    pltpu.sync_copy(ca1_hbm, ca1_v)
    pltpu.sync_copy(cb1_hbm, cb1_v)

    ca0 = plsc.bitcast(ca0_v[...], jnp.uint32)
    cb0 = plsc.bitcast(cb0_v[...], jnp.uint32)
    ca1 = plsc.bitcast(ca1_v[...], jnp.uint32)
    cb1 = plsc.bitcast(cb1_v[...], jnp.uint32)
    iota = _iota()

    def hash_step(p, carry):
        xi = plsc.load_gather(x_v, [2 * p + (iota >> 3)])
        xu = plsc.bitcast(xi, jnp.uint32)
        h0 = plsc.bitcast(_hash_mersenne(xu, ca0, cb0), jnp.int32)
        h1 = plsc.bitcast(_hash_mersenne(xu, ca1, cb1), jnp.int32)
        q = h0 >> RWB
        s = 16 * p + iota
        sl = s & (SLICES_G - 1)
        gbase = (s >> 8) * ROWS_G
        for i in range(RPS):
            plsc.store_scatter(rowidx_v, [gbase + RPS * sl + i], q + i)
        plsc.store_scatter(rowidx_v, [gbase + WOFF + sl], h1 >> RWB)
        meta_v[pl.ds(16 * p, L)] = (
            ((h1 & (RW - 1)) << 16) | (RW * RPS * sl + (h0 & (RW - 1))))
        return carry

    def hash_group(g):
        lax.fori_loop(jnp.int32(g * (SLICES_G // L)),
                      jnp.int32((g + 1) * (SLICES_G // L)), hash_step, 0)

    def start(g, rows_v, sem):
        handles = []
        for k in range(ROWS_G // 128):
            handles.append(pltpu.async_copy(
                table_hbm.at[rowidx_v.at[pl.ds(g * ROWS_G + k * 128, 128)]],
                rows_v.at[pl.ds(k * 128, 128)], sem))
        return handles

    bufs = [(rows_a, sem_a), (rows_b, sem_b), (rows_c, sem_c)]
    out_bufs = [out_a, out_b]
    hash_group(0)
    pend = {0: start(0, *bufs[0])}
    hash_group(1)
    pend[1] = start(1, *bufs[1])
    out_pend = [None, None]

    for g in range(NG):
        b = g % 3
        if g + 2 < NG:
            hash_group(g + 2)
            pend[g + 2] = start(g + 2, *bufs[(g + 2) % 3])
        for h in pend.pop(g):
            h.wait()
        rows_v, _ = bufs[b]
        ob = g & 1
        out_g = out_bufs[ob]
        if out_pend[ob] is not None:
            out_pend[ob].wait()
            out_pend[ob] = None

        def item_step(i, carry2, rows_v=rows_v, out_g=out_g, g=g):
            accs = [jnp.zeros((L,), jnp.float32) for _ in range(DIM // L)]
            for c in range(N_CHUNKS):
                sl = i * N_CHUNKS + c
                meta = plsc.load_gather(
                    meta_v, [jnp.full((L,), g * SLICES_G, jnp.int32) + sl])
                wc = meta >> 16
                base = (meta & 0xFFFF) + iota
                w = plsc.load_gather(
                    rows_v, [jnp.full((L,), WOFF + sl, jnp.int32), wc])
                for v in range(DIM // L):
                    flat = base + v * L
                    vec = plsc.load_gather(rows_v, [flat >> RWB, flat & (RW - 1)])
                    accs[v] = accs[v] + vec * w
            for v in range(DIM // L):
                out_g[i, pl.ds(v * L, L)] = accs[v] * SCALE
            return carry2

        lax.fori_loop(jnp.int32(0), jnp.int32(G), item_step, 0)
        out_pend[ob] = pltpu.async_copy(
            out_g, out_hbm.at[pl.ds(base_b + g * G, G)], sem_out)

    for op in out_pend:
        if op is not None:
            op.wait()


@jax.jit
def _robe(x32, table16, ca0, cb0, ca1, cb1):
    mesh = plsc.VectorSubcoreMesh(core_axis_name="c", subcore_axis_name="s")
    f = pl.kernel(
        _body,
        mesh=mesh,
        out_type=jax.ShapeDtypeStruct((BATCH, DIM), jnp.float32),
        compiler_params=pltpu.CompilerParams(
            needs_layout_passes=False, use_tc_tiling_on_sc=False),
        scratch_types=[
            pltpu.VMEM((B_W,), jnp.int32),
            pltpu.VMEM((L,), jnp.int32),
            pltpu.VMEM((L,), jnp.int32),
            pltpu.VMEM((L,), jnp.int32),
            pltpu.VMEM((L,), jnp.int32),
            pltpu.VMEM((NG * ROWS_G,), jnp.int32),
            pltpu.VMEM((N_SLICES,), jnp.int32),
            pltpu.VMEM((ROWS_G, RW), jnp.float32),
            pltpu.VMEM((ROWS_G, RW), jnp.float32),
            pltpu.VMEM((ROWS_G, RW), jnp.float32),
            pltpu.VMEM((G, DIM), jnp.float32),
            pltpu.VMEM((G, DIM), jnp.float32),
            pltpu.SemaphoreType.DMA,
            pltpu.SemaphoreType.DMA,
            pltpu.SemaphoreType.DMA,
            pltpu.SemaphoreType.DMA,
        ],
    )
    return f(x32, ca0, cb0, ca1, cb1, table16)


def kernel(x, table, coeffs0, coeffs1):
    x32 = x.astype(jnp.int32)
    table16 = table.reshape(VROWS, RW)
    ca0 = jnp.tile(coeffs0[:, 0].astype(jnp.int32), 2)
    cb0 = jnp.tile(coeffs0[:, 1].astype(jnp.int32), 2)
    ca1 = jnp.tile(coeffs1[:, 0].astype(jnp.int32), 2)
    cb1 = jnp.tile(coeffs1[:, 1].astype(jnp.int32), 2)
    return _robe(x32, table16, ca0, cb0, ca1, cb1)
